# baseline (speedup 1.0000x reference)
import jax
import jax.numpy as jnp
from jax.experimental import pallas as pl

_NUM_EXPERTS = 64
_TOP_K = 8
_BLK = 1024


def _router_body(x_ref, bias_ref, idx_ref, w_ref):
    x = x_ref[...] + bias_ref[...]
    iota = jax.lax.broadcasted_iota(jnp.int32, x.shape, 1)
    vals = []
    idxs = []
    for _ in range(_TOP_K):
        m = jnp.max(x, axis=-1, keepdims=True)
        am = jnp.min(jnp.where(x == m, iota, _NUM_EXPERTS), axis=-1,
                     keepdims=True)
        vals.append(m)
        idxs.append(am)
        x = jnp.where(iota == am, -jnp.inf, x)
    v = jnp.concatenate(vals, axis=-1)
    i = jnp.concatenate(idxs, axis=-1)
    e = jnp.exp(v - v[:, 0:1])
    w = e / jnp.sum(e, axis=-1, keepdims=True)
    idx_ref[...] = i
    w_ref[...] = w


@jax.jit
def kernel(gate_logits, bias):
    n, _ = gate_logits.shape
    grid = (n // _BLK,)
    bias2d = bias.reshape(1, _NUM_EXPERTS)
    idx, w = pl.pallas_call(
        _router_body,
        grid=grid,
        in_specs=[
            pl.BlockSpec((_BLK, _NUM_EXPERTS), lambda i: (i, 0)),
            pl.BlockSpec((1, _NUM_EXPERTS), lambda i: (0, 0)),
        ],
        out_specs=[
            pl.BlockSpec((_BLK, _TOP_K), lambda i: (i, 0)),
            pl.BlockSpec((_BLK, _TOP_K), lambda i: (i, 0)),
        ],
        out_shape=[
            jax.ShapeDtypeStruct((n, _TOP_K), jnp.int32),
            jax.ShapeDtypeStruct((n, _TOP_K), jnp.float32),
        ],
    )(gate_logits, bias2d)
    return idx, w
